# Initial kernel scaffold; baseline (speedup 1.0000x reference)
import jax
import jax.numpy as jnp
from jax.experimental import pallas as pl


def kernel(x, table):
    raise NotImplementedError("write your pallas kernel here")


# speedup vs baseline: 1.0345x; 1.0345x over previous
import functools
import math

import jax
import jax.numpy as jnp
from jax import lax
from jax.experimental import pallas as pl
from jax.experimental.pallas import tpu as pltpu
from jax.experimental.pallas import tpu_sc as plsc

D_MODEL = 1024
SCALE = math.sqrt(D_MODEL)
NC, NS, LANES = 2, 16, 16
NW = NC * NS
CHUNK = 32


def _embed_kernel(n_per_w, table_hbm, idx_hbm, out_hbm, idx_v, rows_v, sem):
    wid = lax.axis_index("s") * NC + lax.axis_index("c")
    base = wid * n_per_w
    pltpu.sync_copy(idx_hbm.at[pl.ds(base, n_per_w)], idx_v)

    n_chunks = n_per_w // CHUNK
    vecs_per_row = D_MODEL // LANES

    def chunk_body(c, _):
        pltpu.async_copy(
            table_hbm.at[idx_v.at[pl.ds(c * CHUNK, CHUNK)]], rows_v, sem
        ).wait()

        def row_body(r, _):
            for v in range(vecs_per_row):
                sl = pl.ds(v * LANES, LANES)
                rows_v[r, sl] = rows_v[r, sl] * SCALE
            return 0

        lax.fori_loop(0, CHUNK, row_body, 0)
        pltpu.sync_copy(rows_v, out_hbm.at[pl.ds(base + c * CHUNK, CHUNK)])
        return 0

    lax.fori_loop(0, n_chunks, chunk_body, 0)


@jax.jit
def kernel(x, table):
    B, L = x.shape
    n = B * L
    idx = x.reshape(n).astype(jnp.int32)
    n_per_w = n // NW

    mesh = plsc.VectorSubcoreMesh(
        core_axis_name="c", subcore_axis_name="s", num_cores=NC, num_subcores=NS
    )
    out = pl.kernel(
        functools.partial(_embed_kernel, n_per_w),
        out_type=jax.ShapeDtypeStruct((n, D_MODEL), jnp.float32),
        mesh=mesh,
        scratch_types=[
            pltpu.VMEM((n_per_w,), jnp.int32),
            pltpu.VMEM((CHUNK, D_MODEL), jnp.float32),
            pltpu.SemaphoreType.DMA,
        ],
    )(table, idx)
    return out.reshape(B, L, D_MODEL)


# speedup vs baseline: 1.3511x; 1.3060x over previous
import functools
import math

import jax
import jax.numpy as jnp
from jax import lax
from jax.experimental import pallas as pl
from jax.experimental.pallas import tpu as pltpu
from jax.experimental.pallas import tpu_sc as plsc

D_MODEL = 1024
SCALE = math.sqrt(D_MODEL)
NC, NS, LANES = 2, 16, 16
NW = NC * NS
CHUNK = 32


def _embed_kernel(n_per_w, table_hbm, idx_hbm, out_hbm, idx_v, rows_v,
                  gsem, wsem):
    wid = lax.axis_index("s") * NC + lax.axis_index("c")
    base = wid * n_per_w
    pltpu.sync_copy(idx_hbm.at[pl.ds(base, n_per_w)], idx_v)

    n_chunks = n_per_w // CHUNK

    def gather(c, nb):
        pltpu.async_copy(
            table_hbm.at[idx_v.at[pl.ds(c * CHUNK, CHUNK)]],
            rows_v.at[nb], gsem[nb])

    def wait_gather(nb):
        pltpu.make_async_copy(
            table_hbm.at[idx_v.at[pl.ds(0, CHUNK)]],
            rows_v.at[nb], gsem[nb]).wait()

    def writeback(c, nb):
        pltpu.async_copy(
            rows_v.at[nb], out_hbm.at[pl.ds(base + c * CHUNK, CHUNK)],
            wsem[nb])

    def wait_writeback(c, nb):
        pltpu.make_async_copy(
            rows_v.at[nb], out_hbm.at[pl.ds(base + c * CHUNK, CHUNK)],
            wsem[nb]).wait()

    gather(0, 0)
    for c in range(n_chunks):
        nb = c % 2
        if c + 1 < n_chunks:
            if c >= 1:
                wait_writeback(c - 1, 1 - nb)
            gather(c + 1, 1 - nb)
        wait_gather(nb)

        def row_body(r, _):
            for v in range(D_MODEL // LANES):
                sl = pl.ds(v * LANES, LANES)
                rows_v[nb, r, sl] = rows_v[nb, r, sl] * SCALE
            return 0

        lax.fori_loop(0, CHUNK, row_body, 0)
        writeback(c, nb)

    wait_writeback(n_chunks - 2, n_chunks % 2)
    wait_writeback(n_chunks - 1, (n_chunks - 1) % 2)


@jax.jit
def kernel(x, table):
    B, L = x.shape
    n = B * L
    idx = x.reshape(n).astype(jnp.int32)
    n_per_w = n // NW

    mesh = plsc.VectorSubcoreMesh(
        core_axis_name="c", subcore_axis_name="s", num_cores=NC, num_subcores=NS
    )
    out = pl.kernel(
        functools.partial(_embed_kernel, n_per_w),
        out_type=jax.ShapeDtypeStruct((n, D_MODEL), jnp.float32),
        mesh=mesh,
        scratch_types=[
            pltpu.VMEM((n_per_w,), jnp.int32),
            pltpu.VMEM((2, CHUNK, D_MODEL), jnp.float32),
            [pltpu.SemaphoreType.DMA, pltpu.SemaphoreType.DMA],
            [pltpu.SemaphoreType.DMA, pltpu.SemaphoreType.DMA],
        ],
    )(table, idx)
    return out.reshape(B, L, D_MODEL)


# speedup vs baseline: 1.4021x; 1.0378x over previous
import functools
import math

import jax
import jax.numpy as jnp
from jax import lax
from jax.experimental import pallas as pl
from jax.experimental.pallas import tpu as pltpu
from jax.experimental.pallas import tpu_sc as plsc

D_MODEL = 1024
SCALE = math.sqrt(D_MODEL)
NC, NS, LANES = 2, 16, 16
NW = NC * NS
CHUNK = 32


NBUF = 3


def _embed_kernel(n_per_w, table_hbm, idx_hbm, out_hbm, idx_v, rows_v,
                  gsem, wsem):
    wid = lax.axis_index("s") * NC + lax.axis_index("c")
    base = wid * n_per_w
    pltpu.sync_copy(idx_hbm.at[pl.ds(base, n_per_w)], idx_v)

    n_chunks = n_per_w // CHUNK

    def gather(c, nb):
        pltpu.async_copy(
            table_hbm.at[idx_v.at[pl.ds(c * CHUNK, CHUNK)]],
            rows_v.at[nb], gsem[nb])

    def wait_gather(nb):
        pltpu.make_async_copy(
            table_hbm.at[idx_v.at[pl.ds(0, CHUNK)]],
            rows_v.at[nb], gsem[nb]).wait()

    def writeback(c, nb):
        pltpu.async_copy(
            rows_v.at[nb], out_hbm.at[pl.ds(base + c * CHUNK, CHUNK)],
            wsem[nb])

    def wait_writeback(c, nb):
        pltpu.make_async_copy(
            rows_v.at[nb], out_hbm.at[pl.ds(base + c * CHUNK, CHUNK)],
            wsem[nb]).wait()

    gather(0, 0)
    gather(1, 1)
    for c in range(n_chunks):
        nb = c % NBUF
        if c + 2 < n_chunks:
            if c >= 1:
                wait_writeback(c - 1, (c - 1) % NBUF)
            gather(c + 2, (c + 2) % NBUF)
        wait_gather(nb)

        @plsc.parallel_loop(0, CHUNK)
        def row_body(r):
            for v in range(D_MODEL // LANES):
                sl = pl.ds(v * LANES, LANES)
                rows_v[nb, r, sl] = rows_v[nb, r, sl] * SCALE

        writeback(c, nb)

    for c in range(max(0, n_chunks - 3), n_chunks):
        wait_writeback(c, c % NBUF)


@jax.jit
def kernel(x, table):
    B, L = x.shape
    n = B * L
    idx = x.reshape(n).astype(jnp.int32)
    n_per_w = n // NW

    mesh = plsc.VectorSubcoreMesh(
        core_axis_name="c", subcore_axis_name="s", num_cores=NC, num_subcores=NS
    )
    out = pl.kernel(
        functools.partial(_embed_kernel, n_per_w),
        out_type=jax.ShapeDtypeStruct((n, D_MODEL), jnp.float32),
        mesh=mesh,
        scratch_types=[
            pltpu.VMEM((n_per_w,), jnp.int32),
            pltpu.VMEM((NBUF, CHUNK, D_MODEL), jnp.float32),
            [pltpu.SemaphoreType.DMA] * NBUF,
            [pltpu.SemaphoreType.DMA] * NBUF,
        ],
    )(table, idx)
    return out.reshape(B, L, D_MODEL)
